# baseline (speedup 1.0000x reference)
import functools

import jax
import jax.numpy as jnp
from jax import lax
from jax.experimental import pallas as pl
from jax.experimental.pallas import tpu as pltpu
from jax.experimental.pallas import tpu_sc as plsc

DIM = 32
BATCH = 16384
NROWS = 1000000
NC, NS = 2, 16
NW = NC * NS
B_PER_W = BATCH // NW
NBLK = (NROWS + 127) // 128
BASE_CNT = NBLK // NW
REM = NBLK - BASE_CNT * NW
WIN = 16
N_WIN = (BASE_CNT + 1 + WIN - 1) // WIN
NVREG = BATCH // 16
RING = 64
NTRASH = 16


def _extract_kernel(uidx_hbm, iidx_hbm, ut_hbm, it_hbm,
                    urows_hbm, irows_hbm,
                    idx_v, selu_v, selb_v, blk, wbuf, wbi, dsem, ssem):
    wid = lax.axis_index("s") * NC + lax.axis_index("c")
    start = wid * BASE_CNT + jnp.minimum(wid, REM)
    cnt = BASE_CNT + (wid < REM).astype(jnp.int32)
    lane = lax.iota(jnp.int32, 16)

    for idx_hbm, tab_hbm, rows_hbm in ((uidx_hbm, ut_hbm, urows_hbm),
                                       (iidx_hbm, it_hbm, irows_hbm)):
        pltpu.sync_copy(idx_hbm, idx_v)

        def sel_body(g, cur):
            v = idx_v[pl.ds(g * 16, 16)]
            blkid = v >> 7
            m = (blkid >= start) & (blkid < start + cnt)
            mi = m.astype(jnp.int32)
            pos = cur + plsc.cumsum(mi) - mi
            plsc.store_scatter(selu_v, [pos], v, mask=m)
            plsc.store_scatter(selb_v, [pos], g * 16 + lane, mask=m)
            return cur + plsc.all_reduce_population_count(m)[0]

        n_sel = lax.fori_loop(0, NVREG, sel_body, jnp.int32(0))
        nv = (n_sel + 15) >> 4

        wstate = (jnp.int32(0), jnp.int32(0))
        for k in range(N_WIN):
            ws = jnp.minimum(start + WIN * k, NBLK - WIN)
            o = pl.multiple_of(ws * 128, 128)
            pltpu.async_copy(tab_hbm.at[:, pl.ds(o, WIN * 128)], blk, dsem).wait()
            wsc = ws * 128

            def ext_body(s, c, wsc=wsc):
                wcur, wf = c
                vu = selu_v[pl.ds(s * 16, 16)]
                vb = selb_v[pl.ds(s * 16, 16)]
                vblk = vu >> 7
                m = (vblk >= wsc // 128) & (vblk < wsc // 128 + WIN)
                m = m & ((s * 16 + lane) < n_sel)
                pc = plsc.all_reduce_population_count(m)[0]
                mi = m.astype(jnp.int32)
                pos = (wcur + plsc.cumsum(mi) - mi) & (RING - 1)
                colw = jnp.where(m, vu - wsc, 0)

                @pl.when(pc > 0)
                def _():
                    for j in range(DIM):
                        js = jnp.full((16,), j, jnp.int32)
                        g = plsc.load_gather(blk, [js, colw])
                        plsc.store_scatter(wbuf, [pos, js], g, mask=m)
                    plsc.store_scatter(wbi, [pos >> 4, pos & 15], vb, mask=m)

                wcur = wcur + pc
                fired = (wcur - wf >= 16).astype(jnp.int32)

                @pl.when(fired == 1)
                def _():
                    fr = (wf >> 4) & (RING // 16 - 1)
                    pltpu.async_copy(
                        wbuf.at[pl.ds(fr * 16, 16)],
                        rows_hbm.at[wbi.at[fr]], ssem)

                return wcur, wf + 16 * fired

            wstate = lax.fori_loop(0, nv, ext_body, wstate)

        wcur, wf = wstate
        rem = wcur - wf
        trash = BATCH + lane
        plsc.store_scatter(wbi, [((wcur + lane) >> 4) & (RING // 16 - 1),
                                 (wcur + lane) & 15], trash)
        plsc.store_scatter(wbi, [((wcur + 16 + lane) >> 4) & (RING // 16 - 1),
                                 (wcur + 16 + lane) & 15], trash)

        for t in range(2):
            @pl.when(rem > 16 * t)
            def _(t=t):
                fr = ((wf + 16 * t) >> 4) & (RING // 16 - 1)
                pltpu.async_copy(
                    wbuf.at[pl.ds(fr * 16, 16)],
                    rows_hbm.at[wbi.at[fr]], ssem)

        nflush = (wf >> 4) + (rem > 0).astype(jnp.int32) + (rem > 16).astype(jnp.int32)

        def drain_body(i, c):
            pltpu.make_async_copy(
                rows_hbm.at[pl.ds(0, 16)], wbuf.at[pl.ds(0, 16)], ssem).wait()
            return c

        lax.fori_loop(0, nflush, drain_body, jnp.int32(0))


DOT_CHUNK = 128


def _dot_kernel(urows_hbm, irows_hbm, out_hbm, uv, iv, out_v, sem):
    wid = lax.axis_index("s") * NC + lax.axis_index("c")
    base = wid * B_PER_W
    lane = lax.iota(jnp.int32, 16)

    def chunk_body(c, carry):
        off = base + c * DOT_CHUNK
        pltpu.sync_copy(urows_hbm.at[pl.ds(off, DOT_CHUNK)], uv)
        pltpu.sync_copy(irows_hbm.at[pl.ds(off, DOT_CHUNK)], iv)

        def group_body(g, carry2):
            rows = g * 16 + lane
            acc = jnp.zeros((16,), jnp.float32)
            for j in range(DIM):
                js = jnp.full((16,), j, jnp.int32)
                acc = acc + (plsc.load_gather(uv, [rows, js])
                             * plsc.load_gather(iv, [rows, js]))
            out_v[pl.ds(c * DOT_CHUNK + g * 16, 16)] = acc
            return carry2

        lax.fori_loop(0, DOT_CHUNK // 16, group_body, 0)
        return carry

    lax.fori_loop(0, B_PER_W // DOT_CHUNK, chunk_body, 0)
    pltpu.sync_copy(out_v, out_hbm.at[pl.ds(base, B_PER_W)])


@jax.jit
def _run(user_idx, item_idx, user_table, item_table):
    ut = jnp.swapaxes(user_table, 0, 1)
    it = jnp.swapaxes(item_table, 0, 1)
    mesh = plsc.VectorSubcoreMesh(
        core_axis_name="c", subcore_axis_name="s",
        num_cores=NC, num_subcores=NS)
    params = pltpu.CompilerParams(
        needs_layout_passes=False, use_tc_tiling_on_sc=True)

    extract = functools.partial(
        pl.kernel,
        out_type=(jax.ShapeDtypeStruct((BATCH + NTRASH, 128), jnp.float32),
                  jax.ShapeDtypeStruct((BATCH + NTRASH, 128), jnp.float32)),
        mesh=mesh,
        scratch_types=[
            pltpu.VMEM((BATCH,), jnp.int32),
            pltpu.VMEM((BATCH,), jnp.int32),
            pltpu.VMEM((BATCH,), jnp.int32),
            pltpu.VMEM((DIM, WIN * 128), jnp.float32),
            pltpu.VMEM((RING, 128), jnp.float32),
            pltpu.VMEM((RING // 16, 16), jnp.int32),
            pltpu.SemaphoreType.DMA,
            pltpu.SemaphoreType.DMA,
        ],
        compiler_params=params,
    )(_extract_kernel)
    urows, irows = extract(user_idx, item_idx, ut, it)

    dot = functools.partial(
        pl.kernel,
        out_type=jax.ShapeDtypeStruct((BATCH,), jnp.float32),
        mesh=mesh,
        scratch_types=[
            pltpu.VMEM((DOT_CHUNK, 128), jnp.float32),
            pltpu.VMEM((DOT_CHUNK, 128), jnp.float32),
            pltpu.VMEM((B_PER_W,), jnp.float32),
            pltpu.SemaphoreType.DMA,
        ],
        compiler_params=params,
    )(_dot_kernel)
    return dot(urows, irows)


def kernel(user_idx, item_idx, user_table, item_table):
    return _run(user_idx.astype(jnp.int32), item_idx.astype(jnp.int32),
                user_table, item_table)


# speedup vs baseline: 1.2352x; 1.2352x over previous
import functools

import jax
import jax.numpy as jnp
from jax import lax
from jax.experimental import pallas as pl
from jax.experimental.pallas import tpu as pltpu
from jax.experimental.pallas import tpu_sc as plsc

DIM = 32
BATCH = 16384
NROWS = 1000000
NC = 2
NS = 16
NW = NC * NS
B_PER_W = BATCH // NW
G = 16
N_GROUPS = B_PER_W // G
TAIL = (NROWS // 128) * 128


def _sc_kernel(user_idx_hbm, item_idx_hbm, ut_hbm, it_hbm,
               ut_tail_hbm, it_tail_hbm,
               out_hbm, uidx_v, iidx_v, blk, rows_u, out_v, sem):
    wid = lax.axis_index("s") * NC + lax.axis_index("c")
    base = wid * B_PER_W

    pltpu.sync_copy(user_idx_hbm.at[pl.ds(base, B_PER_W)], uidx_v)
    pltpu.sync_copy(item_idx_hbm.at[pl.ds(base, B_PER_W)], iidx_v)
    pltpu.sync_copy(ut_tail_hbm, blk.at[G])
    pltpu.sync_copy(it_tail_hbm, blk.at[G + 1])

    lane = lax.iota(jnp.int32, G)

    def fetch_blocks(table_hbm, idx16, tail_slot):
        is_tail = idx16 >= TAIL
        off_vec = jnp.where(is_tail, 0, (idx16 >> 7) * 128)
        col_vec = jnp.where(is_tail, idx16 - TAIL, idx16 - off_vec)
        slot_vec = jnp.where(is_tail, tail_slot, lane)
        copies = []
        for l in range(G):
            o = pl.multiple_of(off_vec[l], 128)
            copies.append(pltpu.async_copy(
                table_hbm.at[:, pl.ds(o, 128)], blk.at[l], sem))
        for cp in copies:
            cp.wait()
        return slot_vec, col_vec

    def group_body(g, carry):
        uidx16 = uidx_v[pl.ds(g * G, G)]
        iidx16 = iidx_v[pl.ds(g * G, G)]

        uslot, ucol = fetch_blocks(ut_hbm, uidx16, G)
        for j in range(DIM):
            js = jnp.full((G,), j, jnp.int32)
            gj = plsc.load_gather(blk, [uslot, js, ucol])
            plsc.store_scatter(rows_u, [lane, js], gj)

        islot, icol = fetch_blocks(it_hbm, iidx16, G + 1)
        acc = jnp.zeros((G,), jnp.float32)
        for j in range(DIM):
            js = jnp.full((G,), j, jnp.int32)
            gi = plsc.load_gather(blk, [islot, js, icol])
            gu = plsc.load_gather(rows_u, [lane, js])
            acc = acc + gu * gi
        out_v[pl.ds(g * G, G)] = acc
        return carry

    lax.fori_loop(0, N_GROUPS, group_body, 0)

    pltpu.sync_copy(out_v, out_hbm.at[pl.ds(base, B_PER_W)])


def _pad_tail(table):
    return jnp.swapaxes(jnp.pad(table[TAIL:], ((0, 128 - (NROWS - TAIL)), (0, 0))), 0, 1)


@jax.jit
def _run(user_idx, item_idx, user_table, item_table):
    ut = jnp.swapaxes(user_table, 0, 1)
    it = jnp.swapaxes(item_table, 0, 1)
    mesh = plsc.VectorSubcoreMesh(
        core_axis_name="c", subcore_axis_name="s",
        num_cores=NC, num_subcores=NS)
    kern = functools.partial(
        pl.kernel,
        out_type=jax.ShapeDtypeStruct((BATCH,), jnp.float32),
        mesh=mesh,
        scratch_types=[
            pltpu.VMEM((B_PER_W,), jnp.int32),
            pltpu.VMEM((B_PER_W,), jnp.int32),
            pltpu.VMEM((G + 2, DIM, 128), jnp.float32),
            pltpu.VMEM((G, DIM), jnp.float32),
            pltpu.VMEM((B_PER_W,), jnp.float32),
            pltpu.SemaphoreType.DMA,
        ],
        compiler_params=pltpu.CompilerParams(
            needs_layout_passes=False, use_tc_tiling_on_sc=True),
    )(_sc_kernel)
    return kern(user_idx, item_idx, ut, it,
                _pad_tail(user_table), _pad_tail(item_table))


def kernel(user_idx, item_idx, user_table, item_table):
    return _run(user_idx.astype(jnp.int32), item_idx.astype(jnp.int32),
                user_table, item_table)
